# Initial kernel scaffold; baseline (speedup 1.0000x reference)
import functools

import jax
import jax.numpy as jnp
from jax import lax
from jax.experimental import pallas as pl
from jax.experimental.pallas import tpu as pltpu
from jax.experimental.pallas import tpu_sc as plsc

R = 20000
N = 100000
D = 128
NNZ = 600000
NC = 2
NS = 16
L = 16
NW = NC * NS
RPW = R // NW
RPC = R // NC
CH = 128
ACC_ROWS = RPC + 240
DUMP = RPC + 64

_mesh = plsc.VectorSubcoreMesh(core_axis_name="c", subcore_axis_name="s")


@functools.partial(
    pl.kernel,
    out_type=jax.ShapeDtypeStruct((R, D), jnp.float32),
    mesh=_mesh,
    scratch_types=[
        pltpu.VMEM((CH,), jnp.int32),
        pltpu.VMEM((CH,), jnp.int32),
        pltpu.VMEM((CH,), jnp.int32),
        pltpu.VMEM((CH, D), jnp.float32),
        pltpu.VMEM((48,), jnp.int32),
        pltpu.VMEM_SHARED((ACC_ROWS, D), jnp.float32),
        pltpu.SemaphoreType.DMA,
    ],
)
def _sc_segment_sum(table_hbm, rows_hbm, cols_hbm, bounds_hbm, out_hbm,
                    cidx, rbuf, lrow, gbuf, bvm, acc, sem):
    c = lax.axis_index("c")
    s = lax.axis_index("s")
    wid = c * NS + s
    sc_row0 = c * RPC

    pltpu.sync_copy(bounds_hbm, bvm)
    b0 = bvm[wid]
    b1 = bvm[wid + 1]
    abase = pl.multiple_of((b0 // 8) * 8, 8)
    nch = (b1 - abase + CH - 1) // CH

    zero16 = jnp.zeros((L,), jnp.float32)

    @pl.loop(0, CH)
    def _(i):
        for j in range(D // L):
            gbuf[i, pl.ds(j * L, L)] = zero16

    for p in range(ACC_ROWS // NS // CH):
        pltpu.sync_copy(gbuf, acc.at[pl.ds(s * (ACC_ROWS // NS) + p * CH, CH)])

    plsc.subcore_barrier()

    @pl.loop(0, nch)
    def _(t):
        off = pl.multiple_of(abase + t * CH, 8)
        pltpu.sync_copy(cols_hbm.at[pl.ds(off, CH)], cidx)
        pltpu.sync_copy(rows_hbm.at[pl.ds(off, CH)], rbuf)
        for j in range(CH // L):
            r = rbuf[pl.ds(j * L, L)]
            pos = off + j * L + lax.iota(jnp.int32, L)
            ok = (pos >= b0) & (pos < b1)
            lrow[pl.ds(j * L, L)] = jnp.where(ok, r - sc_row0, DUMP)
        pltpu.async_copy(table_hbm.at[cidx], gbuf, sem).wait()
        pltpu.sync_copy(gbuf, acc.at[lrow], add=True)

    plsc.subcore_barrier()

    for p in range(5):
        lr0 = s * RPW + p * 125
        gr0 = wid * RPW + p * 125
        pltpu.sync_copy(acc.at[pl.ds(lr0, 125)], gbuf.at[pl.ds(0, 125)])
        pltpu.sync_copy(gbuf.at[pl.ds(0, 125)], out_hbm.at[pl.ds(gr0, 125)])


@jax.jit
def kernel(input_matrix, row_indices, col_indices):
    rows = row_indices.astype(jnp.int32)
    cols = col_indices.astype(jnp.int32)
    marks = jnp.arange(0, R + 1, RPW, dtype=jnp.int32)
    bounds = jnp.searchsorted(rows, marks, side="left").astype(jnp.int32)
    bounds = jnp.concatenate([bounds, jnp.zeros((48 - NW - 1,), jnp.int32)])
    rows_p = jnp.concatenate([rows, jnp.full((CH,), R - 1, jnp.int32)])
    cols_p = jnp.concatenate([cols, jnp.zeros((CH,), jnp.int32)])
    return _sc_segment_sum(input_matrix, rows_p, cols_p, bounds)


# speedup vs baseline: 8.9473x; 8.9473x over previous
import functools

import jax
import jax.numpy as jnp
from jax import lax
from jax.experimental import pallas as pl
from jax.experimental.pallas import tpu as pltpu
from jax.experimental.pallas import tpu_sc as plsc

R = 20000
N = 100000
D = 128
NNZ = 600000
NC = 2
NS = 16
L = 16
NW = NC * NS
RPW = R // NW
RPC = R // NC
CH = 128
ACC_ROWS = RPC + 240
DUMP = RPC + 64

_mesh = plsc.VectorSubcoreMesh(core_axis_name="c", subcore_axis_name="s")


@functools.partial(
    pl.kernel,
    out_type=jax.ShapeDtypeStruct((R, D), jnp.float32),
    mesh=_mesh,
    scratch_types=[
        pltpu.VMEM((CH,), jnp.int32),
        pltpu.VMEM((CH,), jnp.int32),
        pltpu.VMEM((CH,), jnp.int32),
        pltpu.VMEM((CH, D), jnp.float32),
        pltpu.VMEM((48,), jnp.int32),
        pltpu.VMEM_SHARED((ACC_ROWS, D), jnp.float32),
        pltpu.SemaphoreType.DMA,
    ],
)
def _sc_segment_sum(table_hbm, rows_hbm, cols_hbm, bounds_hbm, out_hbm,
                    cidx, rbuf, lrow, gbuf, bvm, acc, sem):
    c = lax.axis_index("c")
    s = lax.axis_index("s")
    wid = c * NS + s
    sc_row0 = c * RPC

    pltpu.sync_copy(bounds_hbm, bvm)
    bvec = bvm[pl.ds(wid, L)]
    b0 = bvec[0]
    b1 = bvec[1]
    abase = pl.multiple_of((b0 // 8) * 8, 8)
    nch = (b1 - abase + CH - 1) // CH

    zero16 = jnp.zeros((L,), jnp.float32)

    @pl.loop(0, CH)
    def _(i):
        for j in range(D // L):
            gbuf[i, pl.ds(j * L, L)] = zero16

    for p in range(ACC_ROWS // NS // CH):
        pltpu.sync_copy(gbuf, acc.at[pl.ds(s * (ACC_ROWS // NS) + p * CH, CH)])

    plsc.subcore_barrier()

    @pl.loop(0, nch)
    def _(t):
        off = pl.multiple_of(abase + t * CH, 8)
        pltpu.sync_copy(cols_hbm.at[pl.ds(off, CH)], cidx)
        pltpu.sync_copy(rows_hbm.at[pl.ds(off, CH)], rbuf)
        for j in range(CH // L):
            r = rbuf[pl.ds(j * L, L)]
            pos = off + j * L + lax.iota(jnp.int32, L)
            ok = (pos >= b0) & (pos < b1)
            lrow[pl.ds(j * L, L)] = jnp.where(ok, r - sc_row0, DUMP)
        pltpu.async_copy(table_hbm.at[cidx], gbuf, sem).wait()
        pltpu.sync_copy(gbuf, acc.at[lrow], add=True)

    plsc.subcore_barrier()

    npieces = RPC // CH
    for q in range(npieces // NS + 1):
        p = s + NS * q

        @pl.when(p < npieces)
        def _():
            lr0 = pl.multiple_of(p * CH, 8)
            gr0 = pl.multiple_of(sc_row0 + p * CH, 8)
            pltpu.sync_copy(acc.at[pl.ds(lr0, CH)], gbuf)
            pltpu.sync_copy(gbuf, out_hbm.at[pl.ds(gr0, CH)])

    tail = RPC - npieces * CH

    @pl.when(s == 0)
    def _():
        pltpu.sync_copy(acc.at[pl.ds(npieces * CH, tail)], gbuf.at[pl.ds(0, tail)])
        pltpu.sync_copy(gbuf.at[pl.ds(0, tail)],
                        out_hbm.at[pl.ds(sc_row0 + npieces * CH, tail)])


@jax.jit
def kernel(input_matrix, row_indices, col_indices):
    rows = row_indices.astype(jnp.int32)
    cols = col_indices.astype(jnp.int32)
    marks = jnp.arange(0, R + 1, RPW, dtype=jnp.int32)
    bounds = jnp.searchsorted(rows, marks, side="left").astype(jnp.int32)
    bounds = jnp.concatenate([bounds, jnp.zeros((48 - NW - 1,), jnp.int32)])
    rows_p = jnp.concatenate([rows, jnp.full((CH,), R - 1, jnp.int32)])
    cols_p = jnp.concatenate([cols, jnp.zeros((CH,), jnp.int32)])
    return _sc_segment_sum(input_matrix, rows_p, cols_p, bounds)


# speedup vs baseline: 20.8608x; 2.3315x over previous
import functools

import jax
import jax.numpy as jnp
from jax import lax
from jax.experimental import pallas as pl
from jax.experimental.pallas import tpu as pltpu
from jax.experimental.pallas import tpu_sc as plsc

R = 20000
N = 100000
D = 128
NNZ = 600000
NC = 2
NS = 16
L = 16
NW = NC * NS
RPW = R // NW
RPC = R // NC
CH = 32
DEP = 8
FD = DEP - 2
CPB = 24
IB = CH * CPB
assert CPB % DEP == 0 and FD <= CPB
ACC_ROWS = RPC + 240
DUMP = RPC + 64

_mesh = plsc.VectorSubcoreMesh(core_axis_name="c", subcore_axis_name="s")

_scratch = (
    [pltpu.VMEM((IB,), jnp.int32)] * 4
    + [pltpu.VMEM((CH,), jnp.int32)] * DEP
    + [pltpu.VMEM((CH, D), jnp.float32)] * DEP
    + [pltpu.VMEM((48,), jnp.int32)]
    + [pltpu.VMEM_SHARED((ACC_ROWS, D), jnp.float32)]
    + [pltpu.SemaphoreType.DMA] * (2 * DEP + 2)
)


@functools.partial(
    pl.kernel,
    out_type=jax.ShapeDtypeStruct((R, D), jnp.float32),
    mesh=_mesh,
    scratch_types=_scratch,
)
def _sc_segment_sum(table_hbm, rows_hbm, cols_hbm, bounds_hbm, out_hbm, *sc):
    cbig = sc[0:2]
    rbig = sc[2:4]
    lrow = sc[4:4 + DEP]
    gbuf = sc[4 + DEP:4 + 2 * DEP]
    bvm = sc[4 + 2 * DEP]
    acc = sc[5 + 2 * DEP]
    semg = sc[6 + 2 * DEP:6 + 3 * DEP]
    sems = sc[6 + 3 * DEP:6 + 4 * DEP]
    semi = sc[6 + 4 * DEP:8 + 4 * DEP]

    c = lax.axis_index("c")
    s = lax.axis_index("s")
    wid = c * NS + s
    sc_row0 = c * RPC

    pltpu.sync_copy(bounds_hbm, bvm)
    bvec = bvm[pl.ds(wid, L)]
    b0 = bvec[0]
    b1 = bvec[1]
    abase = pl.multiple_of((b0 // 8) * 8, 8)
    nch = (b1 - abase + CH - 1) // CH

    zero16 = jnp.zeros((L,), jnp.float32)

    @pl.loop(0, CH)
    def _(i):
        for j in range(D // L):
            gbuf[0][i, pl.ds(j * L, L)] = zero16

    for p in range(ACC_ROWS // NS // CH):
        pltpu.sync_copy(gbuf[0], acc.at[pl.ds(s * (ACC_ROWS // NS) + p * CH, CH)])

    plsc.subcore_barrier()

    nbb = (nch + CPB - 1) // CPB

    def _load_block(b, i):
        off = pl.multiple_of(abase + b * IB, 8)
        pltpu.async_copy(cols_hbm.at[pl.ds(off, IB)], cbig[i], semi[i])
        pltpu.async_copy(rows_hbm.at[pl.ds(off, IB)], rbig[i], semi[i])

    def _wait_block(i):
        pltpu.make_async_copy(cols_hbm.at[pl.ds(0, IB)], cbig[i], semi[i]).wait()
        pltpu.make_async_copy(rows_hbm.at[pl.ds(0, IB)], rbig[i], semi[i]).wait()

    def _fire_gather(i, k, g):
        pltpu.async_copy(table_hbm.at[cbig[i].at[pl.ds(k * CH, CH)]],
                         gbuf[g], semg[g])

    def _chunk_body(t, k, i, g):
        off = abase + t * CH
        for j in range(CH // L):
            r = rbig[i][pl.ds(k * CH + j * L, L)]
            pos = off + j * L + lax.iota(jnp.int32, L)
            ok = (pos >= b0) & (pos < b1)
            lrow[g][pl.ds(j * L, L)] = jnp.where(ok, r - sc_row0, DUMP)
        pltpu.make_async_copy(table_hbm.at[cbig[0].at[pl.ds(0, CH)]],
                              gbuf[g], semg[g]).wait()
        pltpu.async_copy(gbuf[g], acc.at[lrow[g]], sems[g], add=True)

    def _wait_scatter(g):
        pltpu.make_async_copy(gbuf[g], acc.at[lrow[g]], sems[g]).wait()

    @pl.when(0 < nch)
    def _():
        off = pl.multiple_of(abase, 8)
        pltpu.sync_copy(cols_hbm.at[pl.ds(off, IB)], cbig[0])
        pltpu.sync_copy(rows_hbm.at[pl.ds(off, IB)], rbig[0])

        @pl.when(1 < nbb)
        def _():
            _load_block(1, 1)

        for f in range(FD):
            @pl.when(f < nch)
            def _():
                _fire_gather(0, f, f)

    @pl.loop(0, nbb, step=2)
    def _(bb):
        for half in range(2):
            b = bb + half
            cur = half
            nxt = 1 - half

            @pl.when(b < nbb)
            def _():
                for k in range(CPB):
                    t = b * CPB + k
                    g = k % DEP
                    g2 = (k + FD) % DEP

                    @pl.when(t < nch)
                    def _():
                        @pl.when(t >= 2)
                        def _():
                            _wait_scatter((k + DEP - 2) % DEP)

                        @pl.when(t + FD < nch)
                        def _():
                            if k + FD < CPB:
                                _fire_gather(cur, k + FD, g2)
                            elif k + FD == CPB:
                                _wait_block(nxt)
                                _fire_gather(nxt, 0, g2)
                            else:
                                _fire_gather(nxt, k + FD - CPB, g2)

                        _chunk_body(t, k, cur, g)

                @pl.when(b + 2 < nbb)
                def _():
                    _load_block(b + 2, cur)

    for back in (2, 1):
        @pl.when(nch >= back)
        def _():
            for m in range(DEP):
                @pl.when((nch - back) % DEP == m)
                def _():
                    _wait_scatter(m)

    plsc.subcore_barrier()

    npieces = RPC // CH
    for q in range(npieces // NS + 1):
        p = s + NS * q

        @pl.when(p < npieces)
        def _():
            lr0 = pl.multiple_of(p * CH, 8)
            gr0 = pl.multiple_of(sc_row0 + p * CH, 8)
            pltpu.sync_copy(acc.at[pl.ds(lr0, CH)], gbuf[0])
            pltpu.sync_copy(gbuf[0], out_hbm.at[pl.ds(gr0, CH)])

    tail = RPC - npieces * CH

    if tail:
        @pl.when(s == 0)
        def _():
            pltpu.sync_copy(acc.at[pl.ds(npieces * CH, tail)],
                            gbuf[0].at[pl.ds(0, tail)])
            pltpu.sync_copy(gbuf[0].at[pl.ds(0, tail)],
                            out_hbm.at[pl.ds(sc_row0 + npieces * CH, tail)])


@jax.jit
def kernel(input_matrix, row_indices, col_indices):
    rows = row_indices.astype(jnp.int32)
    cols = col_indices.astype(jnp.int32)
    marks = jnp.arange(0, R + 1, RPW, dtype=jnp.int32)
    bounds = jnp.searchsorted(rows, marks, side="left").astype(jnp.int32)
    bounds = jnp.concatenate([bounds, jnp.zeros((48 - NW - 1,), jnp.int32)])
    rows_p = jnp.concatenate([rows, jnp.full((IB,), R - 1, jnp.int32)])
    cols_p = jnp.concatenate([cols, jnp.zeros((IB,), jnp.int32)])
    return _sc_segment_sum(input_matrix, rows_p, cols_p, bounds)


# speedup vs baseline: 20.9209x; 1.0029x over previous
import functools

import jax
import jax.numpy as jnp
from jax import lax
from jax.experimental import pallas as pl
from jax.experimental.pallas import tpu as pltpu
from jax.experimental.pallas import tpu_sc as plsc

R = 20000
N = 100000
D = 128
NNZ = 600000
NC = 2
NS = 16
L = 16
NW = NC * NS
RPW = R // NW
RPC = R // NC
CH = 32
DEP = 10
FD = DEP - 2
CPB = 20
IB = CH * CPB
assert CPB % DEP == 0 and FD <= CPB
ACC_ROWS = RPC + 240
DUMP = RPC + 64

_mesh = plsc.VectorSubcoreMesh(core_axis_name="c", subcore_axis_name="s")

_scratch = (
    [pltpu.VMEM((IB,), jnp.int32)] * 4
    + [pltpu.VMEM((CH,), jnp.int32)] * DEP
    + [pltpu.VMEM((CH, D), jnp.float32)] * DEP
    + [pltpu.VMEM((48,), jnp.int32)]
    + [pltpu.VMEM_SHARED((ACC_ROWS, D), jnp.float32)]
    + [pltpu.SemaphoreType.DMA] * (2 * DEP + 2)
)


@functools.partial(
    pl.kernel,
    out_type=jax.ShapeDtypeStruct((R, D), jnp.float32),
    mesh=_mesh,
    scratch_types=_scratch,
)
def _sc_segment_sum(table_hbm, rows_hbm, cols_hbm, bounds_hbm, out_hbm, *sc):
    cbig = sc[0:2]
    rbig = sc[2:4]
    lrow = sc[4:4 + DEP]
    gbuf = sc[4 + DEP:4 + 2 * DEP]
    bvm = sc[4 + 2 * DEP]
    acc = sc[5 + 2 * DEP]
    semg = sc[6 + 2 * DEP:6 + 3 * DEP]
    sems = sc[6 + 3 * DEP:6 + 4 * DEP]
    semi = sc[6 + 4 * DEP:8 + 4 * DEP]

    c = lax.axis_index("c")
    s = lax.axis_index("s")
    wid = c * NS + s
    sc_row0 = c * RPC

    pltpu.sync_copy(bounds_hbm, bvm)
    bvec = bvm[pl.ds(wid, L)]
    b0 = bvec[0]
    b1 = bvec[1]
    abase = pl.multiple_of((b0 // 8) * 8, 8)
    nch = (b1 - abase + CH - 1) // CH

    zero16 = jnp.zeros((L,), jnp.float32)

    @pl.loop(0, CH)
    def _(i):
        for j in range(D // L):
            gbuf[0][i, pl.ds(j * L, L)] = zero16

    for p in range(ACC_ROWS // NS // CH):
        pltpu.sync_copy(gbuf[0], acc.at[pl.ds(s * (ACC_ROWS // NS) + p * CH, CH)])

    plsc.subcore_barrier()

    nbb = (nch + CPB - 1) // CPB

    def _load_block(b, i):
        off = pl.multiple_of(abase + b * IB, 8)
        pltpu.async_copy(cols_hbm.at[pl.ds(off, IB)], cbig[i], semi[i])
        pltpu.async_copy(rows_hbm.at[pl.ds(off, IB)], rbig[i], semi[i])

    def _wait_block(i):
        pltpu.make_async_copy(cols_hbm.at[pl.ds(0, IB)], cbig[i], semi[i]).wait()
        pltpu.make_async_copy(rows_hbm.at[pl.ds(0, IB)], rbig[i], semi[i]).wait()

    def _fire_gather(i, k, g):
        pltpu.async_copy(table_hbm.at[cbig[i].at[pl.ds(k * CH, CH)]],
                         gbuf[g], semg[g])

    def _chunk_body(t, k, i, g):
        off = abase + t * CH
        for j in range(CH // L):
            r = rbig[i][pl.ds(k * CH + j * L, L)]
            pos = off + j * L + lax.iota(jnp.int32, L)
            ok = (pos >= b0) & (pos < b1)
            lrow[g][pl.ds(j * L, L)] = jnp.where(ok, r - sc_row0, DUMP)
        pltpu.make_async_copy(table_hbm.at[cbig[0].at[pl.ds(0, CH)]],
                              gbuf[g], semg[g]).wait()
        pltpu.async_copy(gbuf[g], acc.at[lrow[g]], sems[g], add=True)

    def _wait_scatter(g):
        pltpu.make_async_copy(gbuf[g], acc.at[lrow[g]], sems[g]).wait()

    @pl.when(0 < nch)
    def _():
        off = pl.multiple_of(abase, 8)
        pltpu.sync_copy(cols_hbm.at[pl.ds(off, IB)], cbig[0])
        pltpu.sync_copy(rows_hbm.at[pl.ds(off, IB)], rbig[0])

        @pl.when(1 < nbb)
        def _():
            _load_block(1, 1)

        for f in range(FD):
            @pl.when(f < nch)
            def _():
                _fire_gather(0, f, f)

    @pl.loop(0, nbb, step=2)
    def _(bb):
        for half in range(2):
            b = bb + half
            cur = half
            nxt = 1 - half

            @pl.when(b < nbb)
            def _():
                for k in range(CPB):
                    t = b * CPB + k
                    g = k % DEP
                    g2 = (k + FD) % DEP

                    @pl.when(t < nch)
                    def _():
                        @pl.when(t >= 2)
                        def _():
                            _wait_scatter((k + DEP - 2) % DEP)

                        @pl.when(t + FD < nch)
                        def _():
                            if k + FD < CPB:
                                _fire_gather(cur, k + FD, g2)
                            elif k + FD == CPB:
                                _wait_block(nxt)
                                _fire_gather(nxt, 0, g2)
                            else:
                                _fire_gather(nxt, k + FD - CPB, g2)

                        _chunk_body(t, k, cur, g)

                @pl.when(b + 2 < nbb)
                def _():
                    _load_block(b + 2, cur)

    for back in (2, 1):
        @pl.when(nch >= back)
        def _():
            for m in range(DEP):
                @pl.when((nch - back) % DEP == m)
                def _():
                    _wait_scatter(m)

    plsc.subcore_barrier()

    npieces = RPC // CH
    for q in range(npieces // NS + 1):
        p = s + NS * q

        @pl.when(p < npieces)
        def _():
            lr0 = pl.multiple_of(p * CH, 8)
            gr0 = pl.multiple_of(sc_row0 + p * CH, 8)
            pltpu.sync_copy(acc.at[pl.ds(lr0, CH)], gbuf[0])
            pltpu.sync_copy(gbuf[0], out_hbm.at[pl.ds(gr0, CH)])

    tail = RPC - npieces * CH

    if tail:
        @pl.when(s == 0)
        def _():
            pltpu.sync_copy(acc.at[pl.ds(npieces * CH, tail)],
                            gbuf[0].at[pl.ds(0, tail)])
            pltpu.sync_copy(gbuf[0].at[pl.ds(0, tail)],
                            out_hbm.at[pl.ds(sc_row0 + npieces * CH, tail)])


@jax.jit
def kernel(input_matrix, row_indices, col_indices):
    rows = row_indices.astype(jnp.int32)
    cols = col_indices.astype(jnp.int32)
    marks = jnp.arange(0, R + 1, RPW, dtype=jnp.int32)
    bounds = jnp.searchsorted(rows, marks, side="left").astype(jnp.int32)
    bounds = jnp.concatenate([bounds, jnp.zeros((48 - NW - 1,), jnp.int32)])
    rows_p = jnp.concatenate([rows, jnp.full((IB,), R - 1, jnp.int32)])
    cols_p = jnp.concatenate([cols, jnp.zeros((IB,), jnp.int32)])
    return _sc_segment_sum(input_matrix, rows_p, cols_p, bounds)
